# baseline (speedup 1.0000x reference)
import jax
import jax.numpy as jnp
from jax import lax
from jax.experimental import pallas as pl
from jax.experimental.pallas import tpu as pltpu
from jax.experimental.pallas import tpu_sc as plsc

N_USERS = 10000
N_ITEMS = 40000
N = N_USERS + N_ITEMS
E = 800000
D = 64
H = 32

NTILE = 16
C = 128
E_PAD = 819200
ROWS_TOT = E_PAD // C
ROWS_PER_TILE = ROWS_TOT // NTILE
PHASES = 20
ROWS_PER_PHASE = ROWS_PER_TILE // PHASES
NBUF = 5
N_PAD = 50048
NROW_T = N_PAD // NTILE
ZROWS = 136


def _spmm_body(x0, x1, src_h, dst_h, w_h, out0, out1,
               acc, src2, dst2, wbuf, rows0, rows1, rows2, rows3, rows4,
               stage, zbuf, sg0, sg1, sg2, sg3, sg4):
  cid = lax.axis_index("c")
  sid = lax.axis_index("s")
  rows = (rows0, rows1, rows2, rows3, rows4)
  gsem = (sg0, sg1, sg2, sg3, sg4)

  def compute_chunk(rows_x, j):
    @pl.loop(0, C // 16)
    def _(k):
      wv = wbuf[j, k]
      for l in range(16):
        wb = jnp.broadcast_to(wv[l], (16,))
        e = k * 16 + l
        lo, hi = plsc.unpack(rows_x[e], format=plsc.PackFormat.INTERLEAVED)
        stage[e, 0:16] = lo * wb
        stage[e, 16:32] = hi * wb

  def half(x_ref, out_ref):
    @pl.loop(0, ZROWS)
    def _(i):
      zbuf[i, 0:16] = jnp.zeros((16,), jnp.float32)
      zbuf[i, 16:32] = jnp.zeros((16,), jnp.float32)
    rowbase = sid * NROW_T

    @pl.loop(0, NROW_T // ZROWS)
    def _(i):
      pltpu.sync_copy(zbuf, acc.at[pl.ds(rowbase + i * ZROWS, ZROWS)])
    plsc.subcore_barrier()

    def gather(j, u):
      return pltpu.make_async_copy(x_ref.at[src2.at[j]], rows[u], gsem[u])

    @pl.loop(0, PHASES)
    def _(ph):
      prow = sid * ROWS_PER_TILE + ph * ROWS_PER_PHASE
      pltpu.sync_copy(src_h.at[pl.ds(prow, ROWS_PER_PHASE)], src2)
      pltpu.sync_copy(dst_h.at[pl.ds(prow, ROWS_PER_PHASE)], dst2)
      pltpu.sync_copy(w_h.at[pl.ds(prow, ROWS_PER_PHASE)], wbuf)
      for u in range(NBUF - 1):
        gather(u, u).start()

      @pl.loop(0, ROWS_PER_PHASE // NBUF)
      def _(i):
        for u in range(NBUF):
          j = NBUF * i + u
          gather(j, u).wait()

          @pl.when(j + NBUF - 1 < ROWS_PER_PHASE)
          def _():
            gather(j + NBUF - 1, (u + NBUF - 1) % NBUF).start()
          compute_chunk(rows[u], j)
          pltpu.sync_copy(stage, acc.at[dst2.at[j]], add=True)

    plsc.subcore_barrier()
    pltpu.sync_copy(acc.at[pl.ds(rowbase, NROW_T)],
                    out_ref.at[pl.ds(rowbase, NROW_T)])

  @pl.when(cid == 0)
  def _():
    half(x0, out0)

  @pl.when(cid == 1)
  def _():
    half(x1, out1)


def _spmm(x0, x1, srcr, dstr, w3):
  mesh = plsc.VectorSubcoreMesh(core_axis_name="c", subcore_axis_name="s")
  f = pl.kernel(
      _spmm_body,
      out_type=[jax.ShapeDtypeStruct((N_PAD, H), jnp.float32),
                jax.ShapeDtypeStruct((N_PAD, H), jnp.float32)],
      mesh=mesh,
      compiler_params=pltpu.CompilerParams(use_tc_tiling_on_sc=False,
                                           needs_layout_passes=False),
      scratch_types=[
          pltpu.VMEM_SHARED((N_PAD, H), jnp.float32),
          pltpu.VMEM((ROWS_PER_PHASE, C), jnp.int32),
          pltpu.VMEM((ROWS_PER_PHASE, C), jnp.int32),
          pltpu.VMEM((ROWS_PER_PHASE, C // 16, 16), jnp.float32),
          pltpu.VMEM((C, H), jnp.bfloat16),
          pltpu.VMEM((C, H), jnp.bfloat16),
          pltpu.VMEM((C, H), jnp.bfloat16),
          pltpu.VMEM((C, H), jnp.bfloat16),
          pltpu.VMEM((C, H), jnp.bfloat16),
          pltpu.VMEM((C, H), jnp.float32),
          pltpu.VMEM((ZROWS, H), jnp.float32),
          pltpu.SemaphoreType.DMA,
          pltpu.SemaphoreType.DMA,
          pltpu.SemaphoreType.DMA,
          pltpu.SemaphoreType.DMA,
          pltpu.SemaphoreType.DMA,
      ],
  )
  return f(x0, x1, srcr, dstr, w3)


def _interleave_bf16(x):
  return jnp.stack([x[:, :16], x[:, 16:]], axis=2).reshape(
      N_PAD, 32).astype(jnp.bfloat16)


R_BLK = 6256


def _tc_mid_body(s0, s1, g0, g1, w0, e2_0, e2_1, p_out):
  s = jnp.concatenate([s0[...], s1[...]], axis=1)
  e1 = jnp.dot(s, w0[...].T, preferred_element_type=jnp.float32)
  e2 = jnp.where(e1 >= 0, e1, 0.3 * e1)
  g = jnp.concatenate([g0[...], g1[...]], axis=1)
  p_out[...] = g + e1 + e2
  e2_0[...] = e2[:, :H]
  e2_1[...] = e2[:, H:]


def _tc_mid(s0, s1, g0, g1, w0):
  grid = (N_PAD // R_BLK,)
  half_spec = pl.BlockSpec((R_BLK, H), lambda i: (i, 0))
  return pl.pallas_call(
      _tc_mid_body,
      grid=grid,
      in_specs=[half_spec, half_spec, half_spec, half_spec,
                pl.BlockSpec((D, D), lambda i: (0, 0))],
      out_specs=[half_spec, half_spec,
                 pl.BlockSpec((R_BLK, D), lambda i: (i, 0))],
      out_shape=[jax.ShapeDtypeStruct((N_PAD, H), jnp.float32),
                 jax.ShapeDtypeStruct((N_PAD, H), jnp.float32),
                 jax.ShapeDtypeStruct((N_PAD, D), jnp.float32)],
  )(s0, s1, g0, g1, w0)


def _tc_final_body(p, s0, s1, w2, out):
  s = jnp.concatenate([s0[...], s1[...]], axis=1)
  e3 = jnp.dot(s, w2[...].T, preferred_element_type=jnp.float32)
  out[...] = (p[...] + e3) * 0.25


def _tc_final(p, s0, s1, w2):
  grid = (N_PAD // R_BLK,)
  half_spec = pl.BlockSpec((R_BLK, H), lambda i: (i, 0))
  return pl.pallas_call(
      _tc_final_body,
      grid=grid,
      in_specs=[pl.BlockSpec((R_BLK, D), lambda i: (i, 0)),
                half_spec, half_spec,
                pl.BlockSpec((D, D), lambda i: (0, 0))],
      out_specs=pl.BlockSpec((R_BLK, D), lambda i: (i, 0)),
      out_shape=jax.ShapeDtypeStruct((N_PAD, D), jnp.float32),
  )(p, s0, s1, w2)


def kernel(user_emb, item_emb, edge_index, edge_weight, W0, W2):
  dst = edge_index[0]
  src = edge_index[1]
  pad = E_PAD - E
  srcr = jnp.concatenate([src, jnp.zeros((pad,), jnp.int32)]).reshape(
      ROWS_TOT, C)
  dstr = jnp.concatenate([dst, jnp.zeros((pad,), jnp.int32)]).reshape(
      ROWS_TOT, C)
  w3 = jnp.concatenate([edge_weight, jnp.zeros((pad,), jnp.float32)]
                       ).reshape(ROWS_TOT, C // 16, 16)
  zpad = jnp.zeros((N_PAD - N, H), jnp.float32)
  ego0 = jnp.concatenate([user_emb[:, :H], item_emb[:, :H], zpad], axis=0)
  ego1 = jnp.concatenate([user_emb[:, H:], item_emb[:, H:], zpad], axis=0)

  s1_0, s1_1 = _spmm(_interleave_bf16(ego0), _interleave_bf16(ego1),
                     srcr, dstr, w3)
  e2_0, e2_1, p_sum = _tc_mid(s1_0, s1_1, ego0, ego1, W0)
  s2_0, s2_1 = _spmm(_interleave_bf16(e2_0), _interleave_bf16(e2_1),
                     srcr, dstr, w3)
  out = _tc_final(p_sum, s2_0, s2_1, W2)
  return out[:N_USERS], out[N_USERS:N]


# speedup vs baseline: 1.0640x; 1.0640x over previous
import jax
import jax.numpy as jnp
from jax import lax
from jax.experimental import pallas as pl
from jax.experimental.pallas import tpu as pltpu
from jax.experimental.pallas import tpu_sc as plsc

N_USERS = 10000
N_ITEMS = 40000
N = N_USERS + N_ITEMS
E = 800000
D = 64
H = 32

NTILE = 16
C = 128
E_PAD = 819200
ROWS_TOT = E_PAD // C
ROWS_PER_TILE = ROWS_TOT // NTILE
PHASES = 20
ROWS_PER_PHASE = ROWS_PER_TILE // PHASES
NBUF = 4
N_PAD = 50048
NROW_T = N_PAD // NTILE
ZROWS = 136


def _spmm_body(x0, x1, src_h, dst_h, w_h, out0, out1,
               acc, src2, dst2, wbuf, rows0, rows1, rows2, rows3,
               stage0, stage1, zbuf, sg0, sg1, sg2, sg3, ss0, ss1):
  cid = lax.axis_index("c")
  sid = lax.axis_index("s")
  rows = (rows0, rows1, rows2, rows3)
  gsem = (sg0, sg1, sg2, sg3)
  stages = (stage0, stage1)
  ssem = (ss0, ss1)

  def compute_chunk(rows_x, stage, j):
    jb = jnp.broadcast_to(j, (16,)).astype(jnp.int32)

    @pl.loop(0, C // 16)
    def _(k):
      kb = jnp.broadcast_to(k, (16,)).astype(jnp.int32)
      for l in range(16):
        wb = plsc.load_gather(
            wbuf, [jb, kb, jnp.full((16,), l, jnp.int32)])
        e = k * 16 + l
        lo, hi = plsc.unpack(rows_x[e], format=plsc.PackFormat.INTERLEAVED)
        stage[e, 0:16] = lo * wb
        stage[e, 16:32] = hi * wb

  def half(x_ref, out_ref):
    @pl.loop(0, ZROWS)
    def _(i):
      zbuf[i, 0:16] = jnp.zeros((16,), jnp.float32)
      zbuf[i, 16:32] = jnp.zeros((16,), jnp.float32)
    rowbase = sid * NROW_T

    @pl.loop(0, NROW_T // ZROWS)
    def _(i):
      pltpu.sync_copy(zbuf, acc.at[pl.ds(rowbase + i * ZROWS, ZROWS)])
    plsc.subcore_barrier()

    def gather(j, u):
      return pltpu.make_async_copy(x_ref.at[src2.at[j]], rows[u], gsem[u])

    @pl.loop(0, PHASES)
    def _(ph):
      prow = sid * ROWS_PER_TILE + ph * ROWS_PER_PHASE
      pltpu.sync_copy(src_h.at[pl.ds(prow, ROWS_PER_PHASE)], src2)
      pltpu.sync_copy(dst_h.at[pl.ds(prow, ROWS_PER_PHASE)], dst2)
      pltpu.sync_copy(w_h.at[pl.ds(prow, ROWS_PER_PHASE)], wbuf)
      for u in range(NBUF - 1):
        gather(u, u).start()

      def scat(ss, j):
        return pltpu.make_async_copy(stages[ss], acc.at[dst2.at[j]],
                                     ssem[ss])

      @pl.loop(0, ROWS_PER_PHASE // NBUF)
      def _(i):
        for u in range(NBUF):
          j = NBUF * i + u
          ss = u % 2

          @pl.when(j >= 2)
          def _():
            scat(ss, j).wait()
          gather(j, u).wait()

          @pl.when(j + NBUF - 1 < ROWS_PER_PHASE)
          def _():
            gather(j + NBUF - 1, (u + NBUF - 1) % NBUF).start()
          compute_chunk(rows[u], stages[ss], j)
          scat(ss, j).start(add=True)
      scat(0, 0).wait()
      scat(1, 0).wait()

    plsc.subcore_barrier()
    pltpu.sync_copy(acc.at[pl.ds(rowbase, NROW_T)],
                    out_ref.at[pl.ds(rowbase, NROW_T)])

  @pl.when(cid == 0)
  def _():
    half(x0, out0)

  @pl.when(cid == 1)
  def _():
    half(x1, out1)


def _spmm(x0, x1, srcr, dstr, w3):
  mesh = plsc.VectorSubcoreMesh(core_axis_name="c", subcore_axis_name="s")
  f = pl.kernel(
      _spmm_body,
      out_type=[jax.ShapeDtypeStruct((N_PAD, H), jnp.float32),
                jax.ShapeDtypeStruct((N_PAD, H), jnp.float32)],
      mesh=mesh,
      compiler_params=pltpu.CompilerParams(use_tc_tiling_on_sc=False,
                                           needs_layout_passes=False),
      scratch_types=[
          pltpu.VMEM_SHARED((N_PAD, H), jnp.float32),
          pltpu.VMEM((ROWS_PER_PHASE, C), jnp.int32),
          pltpu.VMEM((ROWS_PER_PHASE, C), jnp.int32),
          pltpu.VMEM((ROWS_PER_PHASE, C // 16, 16), jnp.float32),
          pltpu.VMEM((C, H), jnp.bfloat16),
          pltpu.VMEM((C, H), jnp.bfloat16),
          pltpu.VMEM((C, H), jnp.bfloat16),
          pltpu.VMEM((C, H), jnp.bfloat16),
          pltpu.VMEM((C, H), jnp.float32),
          pltpu.VMEM((C, H), jnp.float32),
          pltpu.VMEM((ZROWS, H), jnp.float32),
          pltpu.SemaphoreType.DMA,
          pltpu.SemaphoreType.DMA,
          pltpu.SemaphoreType.DMA,
          pltpu.SemaphoreType.DMA,
          pltpu.SemaphoreType.DMA,
          pltpu.SemaphoreType.DMA,
      ],
  )
  return f(x0, x1, srcr, dstr, w3)


def _interleave_bf16(x):
  return jnp.stack([x[:, :16], x[:, 16:]], axis=2).reshape(
      N_PAD, 32).astype(jnp.bfloat16)


R_BLK = 6256


def _tc_mid_body(s0, s1, g0, g1, w0, e2_0, e2_1, p_out):
  s = jnp.concatenate([s0[...], s1[...]], axis=1)
  e1 = jnp.dot(s, w0[...].T, preferred_element_type=jnp.float32)
  e2 = jnp.where(e1 >= 0, e1, 0.3 * e1)
  g = jnp.concatenate([g0[...], g1[...]], axis=1)
  p_out[...] = g + e1 + e2
  e2_0[...] = e2[:, :H]
  e2_1[...] = e2[:, H:]


def _tc_mid(s0, s1, g0, g1, w0):
  grid = (N_PAD // R_BLK,)
  half_spec = pl.BlockSpec((R_BLK, H), lambda i: (i, 0))
  return pl.pallas_call(
      _tc_mid_body,
      grid=grid,
      in_specs=[half_spec, half_spec, half_spec, half_spec,
                pl.BlockSpec((D, D), lambda i: (0, 0))],
      out_specs=[half_spec, half_spec,
                 pl.BlockSpec((R_BLK, D), lambda i: (i, 0))],
      out_shape=[jax.ShapeDtypeStruct((N_PAD, H), jnp.float32),
                 jax.ShapeDtypeStruct((N_PAD, H), jnp.float32),
                 jax.ShapeDtypeStruct((N_PAD, D), jnp.float32)],
  )(s0, s1, g0, g1, w0)


def _tc_final_body(p, s0, s1, w2, out):
  s = jnp.concatenate([s0[...], s1[...]], axis=1)
  e3 = jnp.dot(s, w2[...].T, preferred_element_type=jnp.float32)
  out[...] = (p[...] + e3) * 0.25


def _tc_final(p, s0, s1, w2):
  grid = (N_PAD // R_BLK,)
  half_spec = pl.BlockSpec((R_BLK, H), lambda i: (i, 0))
  return pl.pallas_call(
      _tc_final_body,
      grid=grid,
      in_specs=[pl.BlockSpec((R_BLK, D), lambda i: (i, 0)),
                half_spec, half_spec,
                pl.BlockSpec((D, D), lambda i: (0, 0))],
      out_specs=pl.BlockSpec((R_BLK, D), lambda i: (i, 0)),
      out_shape=jax.ShapeDtypeStruct((N_PAD, D), jnp.float32),
  )(p, s0, s1, w2)


def kernel(user_emb, item_emb, edge_index, edge_weight, W0, W2):
  dst = edge_index[0]
  src = edge_index[1]
  pad = E_PAD - E
  srcr = jnp.concatenate([src, jnp.zeros((pad,), jnp.int32)]).reshape(
      ROWS_TOT, C)
  dstr = jnp.concatenate([dst, jnp.zeros((pad,), jnp.int32)]).reshape(
      ROWS_TOT, C)
  w3 = jnp.concatenate([edge_weight, jnp.zeros((pad,), jnp.float32)]
                       ).reshape(ROWS_TOT, C // 16, 16)
  zpad = jnp.zeros((N_PAD - N, H), jnp.float32)
  ego0 = jnp.concatenate([user_emb[:, :H], item_emb[:, :H], zpad], axis=0)
  ego1 = jnp.concatenate([user_emb[:, H:], item_emb[:, H:], zpad], axis=0)

  s1_0, s1_1 = _spmm(_interleave_bf16(ego0), _interleave_bf16(ego1),
                     srcr, dstr, w3)
  e2_0, e2_1, p_sum = _tc_mid(s1_0, s1_1, ego0, ego1, W0)
  s2_0, s2_1 = _spmm(_interleave_bf16(e2_0), _interleave_bf16(e2_1),
                     srcr, dstr, w3)
  out = _tc_final(p_sum, s2_0, s2_1, W2)
  return out[:N_USERS], out[N_USERS:N]


# speedup vs baseline: 1.2631x; 1.1872x over previous
import jax
import jax.numpy as jnp
from jax import lax
from jax.experimental import pallas as pl
from jax.experimental.pallas import tpu as pltpu
from jax.experimental.pallas import tpu_sc as plsc

N_USERS = 10000
N_ITEMS = 40000
N = N_USERS + N_ITEMS
E = 800000
D = 64
H = 32

NTILE = 16
C = 128
E_PAD = 819200
ROWS_TOT = E_PAD // C
ROWS_PER_TILE = ROWS_TOT // NTILE
PHASES = 20
ROWS_PER_PHASE = ROWS_PER_TILE // PHASES
NBUF = 4
N_PAD = 50048
NROW_T = N_PAD // NTILE
ZROWS = 136


def _spmm_body(x0, x1, src_h, dst_h, w_h, out0, out1,
               acc, src2, dst2, wbuf, rows0, rows1, rows2, rows3, zbuf,
               sg0, sg1, sg2, sg3, sm0, sm1):
  cid = lax.axis_index("c")
  sid = lax.axis_index("s")
  rows = (rows0, rows1, rows2, rows3)
  gsem = (sg0, sg1, sg2, sg3)

  def compute_chunk(rows_x, j):
    @pl.loop(0, C // 16)
    def _(k):
      wv = wbuf[j, k]
      for l in range(16):
        wb = jnp.broadcast_to(wv[l], (16,))
        e = k * 16 + l
        rows_x[e, 0:16] = rows_x[e, 0:16] * wb
        rows_x[e, 16:32] = rows_x[e, 16:32] * wb

  def half(x_ref, out_ref):
    @pl.loop(0, ZROWS)
    def _(i):
      zbuf[i, 0:16] = jnp.zeros((16,), jnp.float32)
      zbuf[i, 16:32] = jnp.zeros((16,), jnp.float32)
    rowbase = sid * NROW_T

    @pl.loop(0, NROW_T // ZROWS)
    def _(i):
      pltpu.sync_copy(zbuf, acc.at[pl.ds(rowbase + i * ZROWS, ZROWS)])
    plsc.subcore_barrier()

    def gather(j, u):
      return pltpu.make_async_copy(x_ref.at[src2.at[j]], rows[u], gsem[u])

    @pl.loop(0, PHASES)
    def _(ph):
      prow = sid * ROWS_PER_TILE + ph * ROWS_PER_PHASE
      src_cp = pltpu.make_async_copy(
          src_h.at[pl.ds(prow, ROWS_PER_PHASE)], src2, sm0)
      dst_cp = pltpu.make_async_copy(
          dst_h.at[pl.ds(prow, ROWS_PER_PHASE)], dst2, sm1)
      w_cp = pltpu.make_async_copy(
          w_h.at[pl.ds(prow, ROWS_PER_PHASE)], wbuf, sm1)
      src_cp.start()
      dst_cp.start()
      w_cp.start()
      src_cp.wait()
      for u in range(NBUF - 1):
        gather(u, u).start()
      dst_cp.wait()
      w_cp.wait()

      @pl.loop(0, ROWS_PER_PHASE // NBUF)
      def _(i):
        for u in range(NBUF):
          j = NBUF * i + u
          gather(j, u).wait()

          @pl.when(j + NBUF - 1 < ROWS_PER_PHASE)
          def _():
            gather(j + NBUF - 1, (u + NBUF - 1) % NBUF).start()
          compute_chunk(rows[u], j)
          pltpu.sync_copy(rows[u], acc.at[dst2.at[j]], add=True)

    plsc.subcore_barrier()
    pltpu.sync_copy(acc.at[pl.ds(rowbase, NROW_T)],
                    out_ref.at[pl.ds(rowbase, NROW_T)])

  @pl.when(cid == 0)
  def _():
    half(x0, out0)

  @pl.when(cid == 1)
  def _():
    half(x1, out1)


def _spmm(x0, x1, srcr, dstr, w3):
  mesh = plsc.VectorSubcoreMesh(core_axis_name="c", subcore_axis_name="s")
  f = pl.kernel(
      _spmm_body,
      out_type=[jax.ShapeDtypeStruct((N_PAD, H), jnp.float32),
                jax.ShapeDtypeStruct((N_PAD, H), jnp.float32)],
      mesh=mesh,
      compiler_params=pltpu.CompilerParams(use_tc_tiling_on_sc=False),
      scratch_types=[
          pltpu.VMEM_SHARED((N_PAD, H), jnp.float32),
          pltpu.VMEM((ROWS_PER_PHASE, C), jnp.int32),
          pltpu.VMEM((ROWS_PER_PHASE, C), jnp.int32),
          pltpu.VMEM((ROWS_PER_PHASE, C // 16, 16), jnp.float32),
          pltpu.VMEM((C, H), jnp.float32),
          pltpu.VMEM((C, H), jnp.float32),
          pltpu.VMEM((C, H), jnp.float32),
          pltpu.VMEM((C, H), jnp.float32),
          pltpu.VMEM((ZROWS, H), jnp.float32),
          pltpu.SemaphoreType.DMA,
          pltpu.SemaphoreType.DMA,
          pltpu.SemaphoreType.DMA,
          pltpu.SemaphoreType.DMA,
          pltpu.SemaphoreType.DMA,
          pltpu.SemaphoreType.DMA,
      ],
  )
  return f(x0, x1, srcr, dstr, w3)


R_BLK = 6256


def _tc_mid_body(s0, s1, g0, g1, w0, e2_0, e2_1, p_out):
  s = jnp.concatenate([s0[...], s1[...]], axis=1)
  e1 = jnp.dot(s, w0[...].T, preferred_element_type=jnp.float32)
  e2 = jnp.where(e1 >= 0, e1, 0.3 * e1)
  g = jnp.concatenate([g0[...], g1[...]], axis=1)
  p_out[...] = g + e1 + e2
  e2_0[...] = e2[:, :H]
  e2_1[...] = e2[:, H:]


def _tc_mid(s0, s1, g0, g1, w0):
  grid = (N_PAD // R_BLK,)
  half_spec = pl.BlockSpec((R_BLK, H), lambda i: (i, 0))
  return pl.pallas_call(
      _tc_mid_body,
      grid=grid,
      in_specs=[half_spec, half_spec, half_spec, half_spec,
                pl.BlockSpec((D, D), lambda i: (0, 0))],
      out_specs=[half_spec, half_spec,
                 pl.BlockSpec((R_BLK, D), lambda i: (i, 0))],
      out_shape=[jax.ShapeDtypeStruct((N_PAD, H), jnp.float32),
                 jax.ShapeDtypeStruct((N_PAD, H), jnp.float32),
                 jax.ShapeDtypeStruct((N_PAD, D), jnp.float32)],
  )(s0, s1, g0, g1, w0)


def _tc_final_body(p, s0, s1, w2, out):
  s = jnp.concatenate([s0[...], s1[...]], axis=1)
  e3 = jnp.dot(s, w2[...].T, preferred_element_type=jnp.float32)
  out[...] = (p[...] + e3) * 0.25


def _tc_final(p, s0, s1, w2):
  grid = (N_PAD // R_BLK,)
  half_spec = pl.BlockSpec((R_BLK, H), lambda i: (i, 0))
  return pl.pallas_call(
      _tc_final_body,
      grid=grid,
      in_specs=[pl.BlockSpec((R_BLK, D), lambda i: (i, 0)),
                half_spec, half_spec,
                pl.BlockSpec((D, D), lambda i: (0, 0))],
      out_specs=pl.BlockSpec((R_BLK, D), lambda i: (i, 0)),
      out_shape=jax.ShapeDtypeStruct((N_PAD, D), jnp.float32),
  )(p, s0, s1, w2)


def kernel(user_emb, item_emb, edge_index, edge_weight, W0, W2):
  dst = edge_index[0]
  src = edge_index[1]
  pad = E_PAD - E
  srcr = jnp.concatenate([src, jnp.zeros((pad,), jnp.int32)]).reshape(
      ROWS_TOT, C)
  dstr = jnp.concatenate([dst, jnp.zeros((pad,), jnp.int32)]).reshape(
      ROWS_TOT, C)
  w3 = jnp.concatenate([edge_weight, jnp.zeros((pad,), jnp.float32)]
                       ).reshape(ROWS_TOT, C // 16, 16)
  zpad = jnp.zeros((N_PAD - N, H), jnp.float32)
  ego0 = jnp.concatenate([user_emb[:, :H], item_emb[:, :H], zpad], axis=0)
  ego1 = jnp.concatenate([user_emb[:, H:], item_emb[:, H:], zpad], axis=0)

  s1_0, s1_1 = _spmm(ego0, ego1, srcr, dstr, w3)
  e2_0, e2_1, p_sum = _tc_mid(s1_0, s1_1, ego0, ego1, W0)
  s2_0, s2_1 = _spmm(e2_0, e2_1, srcr, dstr, w3)
  out = _tc_final(p_sum, s2_0, s2_1, W2)
  return out[:N_USERS], out[N_USERS:N]
